# baseline (speedup 1.0000x reference)
import functools

import jax
import jax.numpy as jnp
from jax import lax
from jax.experimental import pallas as pl
from jax.experimental.pallas import tpu as pltpu
from jax.experimental.pallas import tpu_sc as plsc

NUM_HEADS = 16
HEAD_DIM = 128
MEMORY_SIZE = 1000
K_SIMILAR = 8
K_CONTIGUOUS = 2
KK = K_SIMILAR + K_CONTIGUOUS
SURPRISE_THRESHOLD = 0.5
E_WIN = 1024
EPS = 1e-8

_INTERPRET = False


def _s1_body(k_ref, qv_ref, surp_ref, pq_ref):
    h = pl.program_id(1)
    nh = pl.num_programs(1)
    kb = k_ref[0, 0]
    kprev = jnp.concatenate([kb[0:1], kb[:-1]], axis=0)
    d = kb - kprev
    ssq = jnp.sum(d * d, axis=1, keepdims=True)
    qh = qv_ref[:, pl.ds(h, 1), :]
    p0 = jnp.sum(kb * qh[0], axis=1, keepdims=True)
    p1 = jnp.sum(kb * qh[1], axis=1, keepdims=True)
    pcat = jnp.concatenate([p0, p1], axis=1)

    @pl.when(h == 0)
    def _():
        surp_ref[0] = jnp.zeros_like(surp_ref[0])
        pq_ref[0] = jnp.zeros_like(pq_ref[0])

    surp_ref[0] += ssq
    pq_ref[0] += pcat

    @pl.when(h == nh - 1)
    def _():
        surp_ref[0] = jnp.sqrt(surp_ref[0])


def _stage1(k, qv, B, S, D):
    return pl.pallas_call(
        _s1_body,
        grid=(B, NUM_HEADS),
        in_specs=[
            pl.BlockSpec((1, 1, S, D), lambda b, h: (b, h, 0, 0)),
            pl.BlockSpec((B, NUM_HEADS, D), lambda b, h: (0, 0, 0)),
        ],
        out_specs=[
            pl.BlockSpec((1, S, 1), lambda b, h: (b, 0, 0)),
            pl.BlockSpec((1, S, 2), lambda b, h: (b, 0, 0)),
        ],
        out_shape=[
            jax.ShapeDtypeStruct((B, S, 1), jnp.float32),
            jax.ShapeDtypeStruct((B, S, 2), jnp.float32),
        ],
        interpret=_INTERPRET,
    )(k, qv)


def _s2_body(surp_ref, segtail_ref, n_ref, B, S):
    for b in range(B):
        s = surp_ref[b]
        mean = jnp.sum(s) / S
        var = jnp.sum((s - mean) ** 2) / (S - 1)
        thr = mean + SURPRISE_THRESHOLD * jnp.sqrt(var)
        pos = lax.broadcasted_iota(jnp.int32, (S, 1), 0)
        bmask = (s > thr) | (pos == S - 1)
        bint = bmask.astype(jnp.int32)
        x = bint
        sh = 1
        while sh < S:
            x = x + jnp.concatenate(
                [jnp.zeros((sh, 1), jnp.int32), x[: S - sh]], axis=0)
            sh *= 2
        seg = x - bint
        n = jnp.sum(bint)
        st = seg - (n - E_WIN)
        segtail_ref[b] = jnp.where(st < 0, E_WIN, st)
        n_ref[b] = n


def _stage2(surp, B, S):
    return pl.pallas_call(
        functools.partial(_s2_body, B=B, S=S),
        in_specs=[pl.BlockSpec((B, S, 1), lambda: (0, 0, 0))],
        out_specs=[
            pl.BlockSpec((B, S, 1), lambda: (0, 0, 0)),
            pl.BlockSpec(memory_space=pltpu.SMEM),
        ],
        out_shape=[
            jax.ShapeDtypeStruct((B, S, 1), jnp.int32),
            jax.ShapeDtypeStruct((B,), jnp.int32),
        ],
        interpret=_INTERPRET,
    )(surp)


def _sc_segsum(k, segidx, zeros, B, S, D):
    H = NUM_HEADS
    CH = 128
    NCH = S // CH
    EH = E_WIN // 2
    ROWS = EH + 8
    mesh = plsc.VectorSubcoreMesh(core_axis_name="c", subcore_axis_name="s")

    NB = 3

    @functools.partial(
        pl.kernel,
        mesh=mesh,
        out_type=jax.ShapeDtypeStruct((B, H, E_WIN, D), jnp.float32),
        scratch_types=[
            pltpu.VMEM_SHARED((16 * ROWS, D), jnp.float32),
            pltpu.VMEM((NB, CH, D), jnp.float32),
            pltpu.VMEM((NCH, CH), jnp.int32),
            pltpu.SemaphoreType.DMA((NB,)),
            pltpu.SemaphoreType.DMA((NB,)),
        ],
    )
    def body(k_hbm, seg_hbm, z_hbm, out_hbm, shared, buf, idx2d, lsem, ssem):
        s = lax.axis_index("s")
        wid = s * 2 + lax.axis_index("c")
        base = s * ROWS
        for rr in range(2):
            tid = rr * 32 + wid
            b = tid // (H * 2)
            rem = tid % (H * 2)
            h = rem // 2
            lo = (rem % 2) * EH
            pltpu.sync_copy(z_hbm, buf.at[0])
            for p in range(EH // CH):
                pltpu.sync_copy(buf.at[0], shared.at[pl.ds(base + p * CH, CH)])
            pltpu.sync_copy(buf.at[0, pl.ds(0, ROWS - EH)],
                            shared.at[pl.ds(base + EH, ROWS - EH)])
            pltpu.sync_copy(seg_hbm.at[b], idx2d)
            for i in range(NCH):
                for j in range(CH // 16):
                    sl = idx2d[i, pl.ds(j * 16, 16)] - lo
                    sl = jnp.where((sl >= 0) & (sl < EH), sl, EH)
                    idx2d[i, pl.ds(j * 16, 16)] = sl + base
            ld = [None] * NCH
            sc = [None] * NCH
            for ci in range(min(NB, NCH)):
                ld[ci] = pltpu.async_copy(
                    k_hbm.at[b, h, pl.ds(ci * CH, CH), :],
                    buf.at[ci], lsem.at[ci])
            for ci in range(NCH):
                ld[ci].wait()
                sc[ci] = pltpu.async_copy(
                    buf.at[ci % NB], shared.at[idx2d.at[ci]],
                    ssem.at[ci % NB], add=True)
                nxt = ci + NB
                if nxt < NCH:
                    sc[ci].wait()
                    ld[nxt] = pltpu.async_copy(
                        k_hbm.at[b, h, pl.ds(nxt * CH, CH), :],
                        buf.at[nxt % NB], lsem.at[nxt % NB])
            for ci in range(max(NCH - NB, 0), NCH):
                sc[ci].wait()
            pltpu.sync_copy(
                shared.at[pl.ds(base, EH)],
                out_hbm.at[b, h, pl.ds(lo, EH), :])

    return body(k, segidx, zeros)


def _s3b_body(seg_ref, pq_ref, out_ref):
    st = seg_ref[0]
    S = st.shape[0]
    ei = lax.broadcasted_iota(jnp.int32, (S, E_WIN), 1)
    a_t = (ei == st).astype(jnp.float32)
    lanes = lax.broadcasted_iota(jnp.int32, (S, 128), 1)
    p0 = pq_ref[0, :, 0:1]
    p1 = pq_ref[0, :, 1:2]
    cols = jnp.where(
        lanes == 0, 1.0,
        jnp.where(lanes == 1, jnp.broadcast_to(p0, (S, 128)),
                  jnp.where(lanes == 2, jnp.broadcast_to(p1, (S, 128)), 0.0)))
    out_ref[0] = lax.dot_general(
        a_t, cols, (((0,), (0,)), ((), ())),
        preferred_element_type=jnp.float32,
        precision=lax.Precision.HIGHEST)


def _stage3b(segtail, pq, B, S):
    return pl.pallas_call(
        _s3b_body,
        grid=(B,),
        in_specs=[
            pl.BlockSpec((1, S, 1), lambda b: (b, 0, 0)),
            pl.BlockSpec((1, S, 2), lambda b: (b, 0, 0)),
        ],
        out_specs=pl.BlockSpec((1, E_WIN, 128), lambda b: (b, 0, 0)),
        out_shape=jax.ShapeDtypeStruct((B, E_WIN, 128), jnp.float32),
        interpret=_INTERPRET,
    )(segtail, pq)


def _s4_body(ss_ref, norm2_ref):
    h = pl.program_id(1)

    @pl.when(h == 0)
    def _():
        norm2_ref[0] = jnp.zeros_like(norm2_ref[0])

    x = ss_ref[0, 0]
    norm2_ref[0] += jnp.sum(x * x, axis=1, keepdims=True)


def _stage4(segsum, B, D):
    return pl.pallas_call(
        _s4_body,
        grid=(B, NUM_HEADS),
        in_specs=[pl.BlockSpec((1, 1, E_WIN, D), lambda b, h: (b, h, 0, 0))],
        out_specs=pl.BlockSpec((1, E_WIN, 1), lambda b, h: (b, 0, 0)),
        out_shape=jax.ShapeDtypeStruct((B, E_WIN, 1), jnp.float32),
        interpret=_INTERPRET,
    )(segsum)


def _s5_body(norm2_ref, conl_ref, qv_ref, n_ref, win_ref, B):
    E = E_WIN
    n0 = n_ref[0]
    n1 = n_ref[1]
    cap = jnp.int32(MEMORY_SIZE)
    v00 = jnp.minimum(n0, cap)
    L = jnp.maximum(n0 + n1 - cap, 0)
    v10 = jnp.maximum(n0 - L, 0)
    v11 = n1 - jnp.maximum(L - n0, 0)
    eio = lax.broadcasted_iota(jnp.int32, (E, 1), 0)
    gio = lax.broadcasted_iota(jnp.int32, (2 * E, 1), 0)
    neg = jnp.float32(-jnp.inf)
    for r in range(2):
        x = qv_ref[r]
        qn = jnp.maximum(jnp.sqrt(jnp.sum(x * x)), EPS)
        parts = []
        for b in range(2):
            num = conl_ref[b, :, 1 + r:2 + r]
            count = conl_ref[b, :, 0:1]
            norm2 = norm2_ref[b]
            numm = num / count
            nm = jnp.sqrt(norm2) / count
            sims = numm / (jnp.maximum(nm, EPS) * qn)
            v_rb = (v00 if b == 0 else jnp.int32(0)) if r == 0 else (
                v10 if b == 0 else v11)
            valid = eio >= (E - v_rb)
            parts.append(jnp.where(valid, sims, neg))
        svec = jnp.concatenate(parts, axis=0)
        for j in range(KK):
            m = jnp.max(svec)
            cand = jnp.where(svec == m, gio, jnp.int32(2 * E))
            gj = jnp.min(cand)
            win_ref[r, j] = gj
            svec = jnp.where(gio == gj, neg, svec)


def _stage5(norm2, conl, qv, nvec, B):
    return pl.pallas_call(
        functools.partial(_s5_body, B=B),
        in_specs=[
            pl.BlockSpec((B, E_WIN, 1), lambda: (0, 0, 0)),
            pl.BlockSpec((B, E_WIN, 128), lambda: (0, 0, 0)),
            pl.BlockSpec((B, NUM_HEADS, HEAD_DIM), lambda: (0, 0, 0)),
            pl.BlockSpec(memory_space=pltpu.SMEM),
        ],
        out_specs=pl.BlockSpec(memory_space=pltpu.SMEM),
        out_shape=jax.ShapeDtypeStruct((2, KK), jnp.int32),
        interpret=_INTERPRET,
    )(norm2, conl, qv, nvec)


def _s6_body(ss_ref, conl_ref, win_ref, out_ref):
    r = pl.program_id(0)
    out_ref[0, 0] = jnp.zeros_like(out_ref[0, 0])
    for j in range(KK):
        g = win_ref[r, j]
        b = g // E_WIN
        e = g - b * E_WIN
        row = ss_ref[pl.ds(b, 1), 0, pl.ds(e, 1), :]
        cnt = conl_ref[pl.ds(g, 1), 0:1]
        out_ref[0, 0, pl.ds(j, 1), :] = row[0] / cnt


def _stage6(segsum, conl_flat, win, B, D):
    return pl.pallas_call(
        _s6_body,
        grid=(2, NUM_HEADS),
        in_specs=[
            pl.BlockSpec((B, 1, E_WIN, D), lambda r, h: (0, h, 0, 0)),
            pl.BlockSpec((B * E_WIN, 128), lambda r, h: (0, 0)),
            pl.BlockSpec(memory_space=pltpu.SMEM),
        ],
        out_specs=pl.BlockSpec((1, 1, 16, D), lambda r, h: (r, h, 0, 0)),
        out_shape=jax.ShapeDtypeStruct((2, NUM_HEADS, 16, D), jnp.float32),
        interpret=_INTERPRET,
    )(segsum, conl_flat, win)


def kernel(inputs, q, k, v, attention_mask, token_indices, seq_len_q):
    B, H, S, D = k.shape
    qv = k[:, :, S - 1, :]

    surp, pq = _stage1(k, qv, B, S, D)
    segtail, nvec = _stage2(surp, B, S)
    zeros = jnp.zeros((128, D), jnp.float32)
    segsum = _sc_segsum(k, segtail.reshape(B, S // 128, 128), zeros, B, S, D)
    conl = _stage3b(segtail, pq, B, S)
    norm2 = _stage4(segsum, B, D)
    win = _stage5(norm2, conl, qv, nvec, B)
    rkp = _stage6(segsum, conl.reshape(B * E_WIN, 128), win, B, D)
    rk = rkp[:, :, :KK, :]

    ak = jnp.concatenate([rk, k], axis=2)
    av = jnp.concatenate([rk, v], axis=2)
    am = jnp.concatenate(
        [jnp.ones((B, KK), attention_mask.dtype), attention_mask], axis=1)
    cur = token_indices[:, -1]
    rpos = jax.vmap(lambda c: jnp.linspace(c - KK, c - 1, KK))(cur)
    ap = jnp.concatenate([rpos, token_indices.astype(rpos.dtype)], axis=1)
    return (inputs, q, ak, av, am, token_indices, KK + S, ap)


# speedup vs baseline: 1.0836x; 1.0836x over previous
import functools

import jax
import jax.numpy as jnp
from jax import lax
from jax.experimental import pallas as pl
from jax.experimental.pallas import tpu as pltpu
from jax.experimental.pallas import tpu_sc as plsc

NUM_HEADS = 16
HEAD_DIM = 128
MEMORY_SIZE = 1000
K_SIMILAR = 8
K_CONTIGUOUS = 2
KK = K_SIMILAR + K_CONTIGUOUS
SURPRISE_THRESHOLD = 0.5
E_WIN = 1024
EPS = 1e-8

_INTERPRET = False


def _s1_body(k_ref, qv_ref, surp_ref, pq_ref):
    h = pl.program_id(1)
    nh = pl.num_programs(1)
    kb = k_ref[0, 0]
    kprev = jnp.concatenate([kb[0:1], kb[:-1]], axis=0)
    d = kb - kprev
    ssq = jnp.sum(d * d, axis=1, keepdims=True)
    qh = qv_ref[:, pl.ds(h, 1), :]
    p0 = jnp.sum(kb * qh[0], axis=1, keepdims=True)
    p1 = jnp.sum(kb * qh[1], axis=1, keepdims=True)
    pcat = jnp.concatenate([p0, p1], axis=1)

    @pl.when(h == 0)
    def _():
        surp_ref[0] = jnp.zeros_like(surp_ref[0])
        pq_ref[0] = jnp.zeros_like(pq_ref[0])

    surp_ref[0] += ssq
    pq_ref[0] += pcat

    @pl.when(h == nh - 1)
    def _():
        surp_ref[0] = jnp.sqrt(surp_ref[0])


def _stage1(k, qv, B, S, D):
    return pl.pallas_call(
        _s1_body,
        grid=(B, NUM_HEADS),
        in_specs=[
            pl.BlockSpec((1, 1, S, D), lambda b, h: (b, h, 0, 0)),
            pl.BlockSpec((B, NUM_HEADS, D), lambda b, h: (0, 0, 0)),
        ],
        out_specs=[
            pl.BlockSpec((1, S, 1), lambda b, h: (b, 0, 0)),
            pl.BlockSpec((1, S, 2), lambda b, h: (b, 0, 0)),
        ],
        out_shape=[
            jax.ShapeDtypeStruct((B, S, 1), jnp.float32),
            jax.ShapeDtypeStruct((B, S, 2), jnp.float32),
        ],
        interpret=_INTERPRET,
    )(k, qv)


def _s2_body(surp_ref, segtail_ref, n_ref, cb_ref, B, S):
    for b in range(B):
        s = surp_ref[b]
        mean = jnp.sum(s) / S
        var = jnp.sum((s - mean) ** 2) / (S - 1)
        thr = mean + SURPRISE_THRESHOLD * jnp.sqrt(var)
        pos = lax.broadcasted_iota(jnp.int32, (S, 1), 0)
        bmask = (s > thr) | (pos == S - 1)
        bint = bmask.astype(jnp.int32)
        x = bint
        sh = 1
        while sh < S:
            x = x + jnp.concatenate(
                [jnp.zeros((sh, 1), jnp.int32), x[: S - sh]], axis=0)
            sh *= 2
        seg = x - bint
        n = jnp.sum(bint)
        st = seg - (n - E_WIN)
        segtail_ref[b] = jnp.where(st < 0, E_WIN, st)
        n_ref[b] = n
        oldrows = jnp.sum((st < 0).astype(jnp.int32))
        eh = E_WIN // 2
        for half in range(2):
            lo = half * eh
            cl = jnp.sum(((st >= 0) & (st < lo)).astype(jnp.int32))
            ch = jnp.sum(((st >= 0) & (st < lo + eh)).astype(jnp.int32))
            r0 = oldrows + cl
            r1 = oldrows + ch
            cb_ref[(b * 2 + half) * 16] = r0 // 128
            cb_ref[(b * 2 + half) * 16 + 1] = (r1 + 127) // 128
    for sl16 in range(2 * B):
        for ii in range(2, 16):
            cb_ref[sl16 * 16 + ii] = 0


def _stage2(surp, B, S):
    return pl.pallas_call(
        functools.partial(_s2_body, B=B, S=S),
        in_specs=[pl.BlockSpec((B, S, 1), lambda: (0, 0, 0))],
        out_specs=[
            pl.BlockSpec((B, S, 1), lambda: (0, 0, 0)),
            pl.BlockSpec(memory_space=pltpu.SMEM),
            pl.BlockSpec(memory_space=pltpu.SMEM),
        ],
        out_shape=[
            jax.ShapeDtypeStruct((B, S, 1), jnp.int32),
            jax.ShapeDtypeStruct((B,), jnp.int32),
            jax.ShapeDtypeStruct((64,), jnp.int32),
        ],
        interpret=_INTERPRET,
    )(surp)


def _sc_segsum(k, segidx, zeros, cb, B, S, D):
    H = NUM_HEADS
    CH = 128
    NCH = S // CH
    EH = E_WIN // 2
    ROWS = EH + 8
    mesh = plsc.VectorSubcoreMesh(core_axis_name="c", subcore_axis_name="s")

    NB = 2

    @functools.partial(
        pl.kernel,
        mesh=mesh,
        out_type=jax.ShapeDtypeStruct((B, H, E_WIN, D), jnp.float32),
        scratch_types=[
            pltpu.VMEM_SHARED((16 * ROWS, D), jnp.float32),
            pltpu.VMEM((NB, CH, D), jnp.float32),
            pltpu.VMEM((NCH, CH), jnp.int32),
            pltpu.VMEM((16,), jnp.int32),
            pltpu.SemaphoreType.DMA((NB,)),
            pltpu.SemaphoreType.DMA((NB,)),
        ],
    )
    def body(k_hbm, seg_hbm, z_hbm, cb_hbm, out_hbm, shared, buf, idx2d,
             cbv, lsem, ssem):
        s = lax.axis_index("s")
        wid = s * 2 + lax.axis_index("c")
        base = s * ROWS
        for rr in range(2):
            tid = rr * 32 + wid
            b = tid // (H * 2)
            rem = tid % (H * 2)
            h = rem // 2
            lo = (rem % 2) * EH
            pltpu.sync_copy(z_hbm, buf.at[0])
            for p in range(EH // CH):
                pltpu.sync_copy(buf.at[0], shared.at[pl.ds(base + p * CH, CH)])
            pltpu.sync_copy(buf.at[0, pl.ds(0, ROWS - EH)],
                            shared.at[pl.ds(base + EH, ROWS - EH)])
            pltpu.sync_copy(seg_hbm.at[b], idx2d)
            for i in range(NCH):
                for j in range(CH // 16):
                    slr = idx2d[i, pl.ds(j * 16, 16)] - lo
                    slr = jnp.where((slr >= 0) & (slr < EH), slr, EH)
                    idx2d[i, pl.ds(j * 16, 16)] = slr + base
            pltpu.sync_copy(
                cb_hbm.at[pl.ds((b * 2 + rem % 2) * 16, 16)], cbv)
            cv = cbv[...]
            c0 = cv[0]
            c1 = cv[1]
            ld = [
                pltpu.make_async_copy(
                    k_hbm.at[b, h, pl.ds(ci * CH, CH), :],
                    buf.at[ci % NB], lsem.at[ci % NB])
                for ci in range(NCH)
            ]
            inr = [(c0 <= ci) & (ci < c1) for ci in range(NCH)]

            @pl.when(inr[0])
            def _():
                ld[0].start()

            for ci in range(NCH):
                if ci + 1 < NCH:
                    @pl.when(inr[ci + 1])
                    def _(ci=ci):
                        ld[ci + 1].start()

                @pl.when(inr[ci])
                def _(ci=ci):
                    ld[ci].wait()
                    pltpu.sync_copy(
                        buf.at[ci % NB], shared.at[idx2d.at[ci]], add=True)

            pltpu.sync_copy(
                shared.at[pl.ds(base, EH)],
                out_hbm.at[b, h, pl.ds(lo, EH), :])

    return body(k, segidx, zeros, cb)


def _s3b_body(seg_ref, pq_ref, out_ref):
    st = seg_ref[0]
    S = st.shape[0]
    ei = lax.broadcasted_iota(jnp.int32, (S, E_WIN), 1)
    a_t = (ei == st).astype(jnp.float32)
    lanes = lax.broadcasted_iota(jnp.int32, (S, 128), 1)
    p0 = pq_ref[0, :, 0:1]
    p1 = pq_ref[0, :, 1:2]
    cols = jnp.where(
        lanes == 0, 1.0,
        jnp.where(lanes == 1, jnp.broadcast_to(p0, (S, 128)),
                  jnp.where(lanes == 2, jnp.broadcast_to(p1, (S, 128)), 0.0)))
    out_ref[0] = lax.dot_general(
        a_t, cols, (((0,), (0,)), ((), ())),
        preferred_element_type=jnp.float32,
        precision=lax.Precision.HIGHEST)


def _stage3b(segtail, pq, B, S):
    return pl.pallas_call(
        _s3b_body,
        grid=(B,),
        in_specs=[
            pl.BlockSpec((1, S, 1), lambda b: (b, 0, 0)),
            pl.BlockSpec((1, S, 2), lambda b: (b, 0, 0)),
        ],
        out_specs=pl.BlockSpec((1, E_WIN, 128), lambda b: (b, 0, 0)),
        out_shape=jax.ShapeDtypeStruct((B, E_WIN, 128), jnp.float32),
        interpret=_INTERPRET,
    )(segtail, pq)


def _s4_body(ss_ref, norm2_ref):
    h = pl.program_id(1)

    @pl.when(h == 0)
    def _():
        norm2_ref[0] = jnp.zeros_like(norm2_ref[0])

    x = ss_ref[0, 0]
    norm2_ref[0] += jnp.sum(x * x, axis=1, keepdims=True)


def _stage4(segsum, B, D):
    return pl.pallas_call(
        _s4_body,
        grid=(B, NUM_HEADS),
        in_specs=[pl.BlockSpec((1, 1, E_WIN, D), lambda b, h: (b, h, 0, 0))],
        out_specs=pl.BlockSpec((1, E_WIN, 1), lambda b, h: (b, 0, 0)),
        out_shape=jax.ShapeDtypeStruct((B, E_WIN, 1), jnp.float32),
        interpret=_INTERPRET,
    )(segsum)


def _s5_body(norm2_ref, conl_ref, qv_ref, n_ref, win_ref, B):
    E = E_WIN
    n0 = n_ref[0]
    n1 = n_ref[1]
    cap = jnp.int32(MEMORY_SIZE)
    v00 = jnp.minimum(n0, cap)
    L = jnp.maximum(n0 + n1 - cap, 0)
    v10 = jnp.maximum(n0 - L, 0)
    v11 = n1 - jnp.maximum(L - n0, 0)
    eio = lax.broadcasted_iota(jnp.int32, (E, 1), 0)
    gio = lax.broadcasted_iota(jnp.int32, (2 * E, 1), 0)
    neg = jnp.float32(-jnp.inf)
    for r in range(2):
        x = qv_ref[r]
        qn = jnp.maximum(jnp.sqrt(jnp.sum(x * x)), EPS)
        parts = []
        for b in range(2):
            num = conl_ref[b, :, 1 + r:2 + r]
            count = conl_ref[b, :, 0:1]
            norm2 = norm2_ref[b]
            numm = num / count
            nm = jnp.sqrt(norm2) / count
            sims = numm / (jnp.maximum(nm, EPS) * qn)
            v_rb = (v00 if b == 0 else jnp.int32(0)) if r == 0 else (
                v10 if b == 0 else v11)
            valid = eio >= (E - v_rb)
            parts.append(jnp.where(valid, sims, neg))
        svec = jnp.concatenate(parts, axis=0)
        for j in range(KK):
            m = jnp.max(svec)
            cand = jnp.where(svec == m, gio, jnp.int32(2 * E))
            gj = jnp.min(cand)
            win_ref[r, j] = gj
            svec = jnp.where(gio == gj, neg, svec)


def _stage5(norm2, conl, qv, nvec, B):
    return pl.pallas_call(
        functools.partial(_s5_body, B=B),
        in_specs=[
            pl.BlockSpec((B, E_WIN, 1), lambda: (0, 0, 0)),
            pl.BlockSpec((B, E_WIN, 128), lambda: (0, 0, 0)),
            pl.BlockSpec((B, NUM_HEADS, HEAD_DIM), lambda: (0, 0, 0)),
            pl.BlockSpec(memory_space=pltpu.SMEM),
        ],
        out_specs=pl.BlockSpec(memory_space=pltpu.SMEM),
        out_shape=jax.ShapeDtypeStruct((2, KK), jnp.int32),
        interpret=_INTERPRET,
    )(norm2, conl, qv, nvec)


def _s6_body(ss_ref, conl_ref, win_ref, out_ref):
    r = pl.program_id(0)
    out_ref[0, 0] = jnp.zeros_like(out_ref[0, 0])
    for j in range(KK):
        g = win_ref[r, j]
        b = g // E_WIN
        e = g - b * E_WIN
        row = ss_ref[pl.ds(b, 1), 0, pl.ds(e, 1), :]
        cnt = conl_ref[pl.ds(g, 1), 0:1]
        out_ref[0, 0, pl.ds(j, 1), :] = row[0] / cnt


def _stage6(segsum, conl_flat, win, B, D):
    return pl.pallas_call(
        _s6_body,
        grid=(2, NUM_HEADS),
        in_specs=[
            pl.BlockSpec((B, 1, E_WIN, D), lambda r, h: (0, h, 0, 0)),
            pl.BlockSpec((B * E_WIN, 128), lambda r, h: (0, 0)),
            pl.BlockSpec(memory_space=pltpu.SMEM),
        ],
        out_specs=pl.BlockSpec((1, 1, 16, D), lambda r, h: (r, h, 0, 0)),
        out_shape=jax.ShapeDtypeStruct((2, NUM_HEADS, 16, D), jnp.float32),
        interpret=_INTERPRET,
    )(segsum, conl_flat, win)


def kernel(inputs, q, k, v, attention_mask, token_indices, seq_len_q):
    B, H, S, D = k.shape
    qv = k[:, :, S - 1, :]

    surp, pq = _stage1(k, qv, B, S, D)
    segtail, nvec, cb = _stage2(surp, B, S)
    zeros = jnp.zeros((128, D), jnp.float32)
    segsum = _sc_segsum(
        k, segtail.reshape(B, S // 128, 128), zeros, cb, B, S, D)
    conl = _stage3b(segtail, pq, B, S)
    norm2 = _stage4(segsum, B, D)
    win = _stage5(norm2, conl, qv, nvec, B)
    rkp = _stage6(segsum, conl.reshape(B * E_WIN, 128), win, B, D)
    rk = rkp[:, :, :KK, :]

    ak = jnp.concatenate([rk, k], axis=2)
    av = jnp.concatenate([rk, v], axis=2)
    am = jnp.concatenate(
        [jnp.ones((B, KK), attention_mask.dtype), attention_mask], axis=1)
    cur = token_indices[:, -1]
    rpos = jax.vmap(lambda c: jnp.linspace(c - KK, c - 1, KK))(cur)
    ap = jnp.concatenate([rpos, token_indices.astype(rpos.dtype)], axis=1)
    return (inputs, q, ak, av, am, token_indices, KK + S, ap)
